# baseline (speedup 1.0000x reference)
import functools

import jax
import jax.numpy as jnp
from jax import lax
from jax.experimental import pallas as pl
from jax.experimental.pallas import tpu as pltpu
from jax.experimental.pallas import tpu_sc as plsc

N = 10000
NP = 10112
IN_C = 128
HID = 128
OUT_C = 64
E = 320000

NC = 2
NS = 16
NW = NC * NS
LANES = 128

_mesh = plsc.VectorSubcoreMesh(core_axis_name="c", subcore_axis_name="s")


def _hist_kernel(n_outer, n_inner):
    @functools.partial(
        pl.kernel,
        mesh=_mesh,
        out_type=jax.ShapeDtypeStruct((NC, NP), jnp.float32),
        scratch_types=[
            pltpu.VMEM((n_inner, LANES), jnp.int32),
            pltpu.VMEM((LANES,), jnp.float32),
            pltpu.VMEM_SHARED((NP,), jnp.float32),
            pltpu.VMEM((NP,), jnp.float32),
        ],
    )
    def k(dst2d, out, didx_v, ones_v, hist_sh, z_v):
        c = lax.axis_index("c")
        s = lax.axis_index("s")
        wid = c * NS + s
        for j in range(LANES // 16):
            ones_v[pl.ds(j * 16, 16)] = jnp.ones((16,), jnp.float32)

        @pl.when(s == 0)
        def _():
            def zb(i, _):
                z_v[pl.ds(i * 16, 16)] = jnp.zeros((16,), jnp.float32)
                return _
            lax.fori_loop(0, NP // 16, zb, 0)
            pltpu.sync_copy(z_v, hist_sh)

        plsc.subcore_barrier()

        row_base = wid * (n_outer * n_inner)

        def outer(o, _):
            pltpu.sync_copy(dst2d.at[pl.ds(row_base + o * n_inner, n_inner)],
                            didx_v)
            for j in range(n_inner):
                pltpu.sync_copy(ones_v, hist_sh.at[didx_v.at[j]], add=True)
            return _

        lax.fori_loop(0, n_outer, outer, 0)
        plsc.subcore_barrier()

        @pl.when(s == 0)
        def _():
            pltpu.sync_copy(hist_sh, out.at[c])

    return k


def _agg_kernel(C, n_blocks, nb):
    @functools.partial(
        pl.kernel,
        mesh=_mesh,
        out_type=jax.ShapeDtypeStruct((NC, NP, C), jnp.float32),
        scratch_types=[
            pltpu.VMEM((nb, LANES), jnp.int32),
            pltpu.VMEM((nb, LANES), jnp.int32),
            pltpu.VMEM((nb, LANES), jnp.int32),
            pltpu.VMEM((nb, LANES), jnp.int32),
            pltpu.VMEM((LANES, C), jnp.float32),
            pltpu.VMEM((LANES, C), jnp.float32),
            pltpu.VMEM_SHARED((NP, C), jnp.float32),
            pltpu.SemaphoreType.DMA,
            pltpu.SemaphoreType.DMA,
            pltpu.SemaphoreType.DMA,
            pltpu.SemaphoreType.DMA,
            pltpu.SemaphoreType.DMA,
            pltpu.SemaphoreType.DMA,
        ],
    )
    def k(g_hbm, src2d, dst2d, out,
          sidxA, didxA, sidxB, didxB, rows0, rows1, acc_sh,
          gs0, gs1, ss0, ss1, isA, isB):
        c = lax.axis_index("c")
        s = lax.axis_index("s")
        wid = c * NS + s
        rows_pt = NP // NS
        row0 = wid * n_blocks * nb

        def zrow(i, carry):
            r = i // (C // 16)
            col = (i % (C // 16)) * 16
            rows0[r, pl.ds(col, 16)] = jnp.zeros((16,), jnp.float32)
            return carry
        lax.fori_loop(0, LANES * (C // 16), zrow, 0)
        for kchunk in range(rows_pt // LANES):
            pltpu.sync_copy(
                rows0, acc_sh.at[pl.ds(s * rows_pt + kchunk * LANES, LANES)])
        rem = rows_pt % LANES
        if rem:
            pltpu.sync_copy(
                rows0.at[pl.ds(0, rem)],
                acc_sh.at[pl.ds(s * rows_pt + (rows_pt // LANES) * LANES,
                                rem)])

        def idx_load(bi, sidx, didx, isem):
            pltpu.async_copy(src2d.at[pl.ds(row0 + bi * nb, nb)], sidx, isem)
            pltpu.async_copy(dst2d.at[pl.ds(row0 + bi * nb, nb)], didx, isem)

        def idx_wait(sidx, didx, isem):
            pltpu.make_async_copy(src2d.at[pl.ds(0, nb)], sidx, isem).wait()
            pltpu.make_async_copy(dst2d.at[pl.ds(0, nb)], didx, isem).wait()

        idx_load(0, sidxA, didxA, isA)
        idx_load(1, sidxB, didxB, isB)
        plsc.subcore_barrier()

        def wait_g(rows, sem):
            pltpu.make_async_copy(g_hbm.at[sidxA.at[0]], rows, sem).wait()

        def wait_s(rows, sem):
            pltpu.make_async_copy(rows, acc_sh.at[didxA.at[0]], sem).wait()

        def run_block(sidx, didx):
            pltpu.async_copy(g_hbm.at[sidx.at[0]], rows0, gs0)

            def body(kk, inner):
                e0 = 2 * kk
                wait_g(rows0, gs0)

                @pl.when(kk >= 1)
                def _w1():
                    wait_s(rows1, ss1)
                pltpu.async_copy(g_hbm.at[sidx.at[e0 + 1]], rows1, gs1)
                pltpu.async_copy(rows0, acc_sh.at[didx.at[e0]], ss0,
                                 add=True)

                wait_g(rows1, gs1)
                wait_s(rows0, ss0)

                @pl.when(kk < nb // 2 - 1)
                def _g0():
                    pltpu.async_copy(g_hbm.at[sidx.at[e0 + 2]], rows0, gs0)
                pltpu.async_copy(rows1, acc_sh.at[didx.at[e0 + 1]], ss1,
                                 add=True)
                return inner

            lax.fori_loop(0, nb // 2, body, 0)
            wait_s(rows1, ss1)

        def blk2(m, carry):
            idx_wait(sidxA, didxA, isA)
            run_block(sidxA, didxA)

            @pl.when(m < n_blocks // 2 - 1)
            def _pa():
                idx_load(2 * m + 2, sidxA, didxA, isA)

            idx_wait(sidxB, didxB, isB)
            run_block(sidxB, didxB)

            @pl.when(m < n_blocks // 2 - 1)
            def _pb():
                idx_load(2 * m + 3, sidxB, didxB, isB)
            return carry

        lax.fori_loop(0, n_blocks // 2, blk2, 0)
        plsc.subcore_barrier()

        pltpu.sync_copy(acc_sh.at[pl.ds(s * rows_pt, rows_pt)],
                        out.at[c, pl.ds(s * rows_pt, rows_pt)])

    return k


def _score_kernel(n_outer):
    FPP = 4
    blocks_pt = n_outer * 16

    @functools.partial(
        pl.kernel,
        mesh=_mesh,
        out_type=jax.ShapeDtypeStruct((NC * 2 * blocks_pt, LANES),
                                      jnp.float32),
        compiler_params=pltpu.CompilerParams(needs_layout_passes=False),
        scratch_types=[
            pltpu.VMEM((FPP * NP,), jnp.int32),
            pltpu.VMEM((16, LANES), jnp.int32),
            pltpu.VMEM((16, LANES), jnp.int32),
            pltpu.VMEM((16, LANES), jnp.float32),
            pltpu.VMEM_SHARED((2 * blocks_pt, LANES), jnp.float32),
        ],
    )
    def k(z2t_hbm, src2d, dst2d, out,
          slab, sidx_v, didx_v, scores_v, score_sh):
        c = lax.axis_index("c")
        s = lax.axis_index("s")
        grp = s // 8
        frow = (s % 8) * FPP

        for j in range(16):
            for l in range(LANES // 16):
                scores_v[j, pl.ds(l * 16, 16)] = jnp.zeros((16,), jnp.float32)
        rows_sh = 2 * blocks_pt
        rpt = rows_sh // NS
        def zb(i, _):
            pltpu.sync_copy(scores_v, score_sh.at[pl.ds(s * rpt + i * 16, 16)])
            return _
        lax.fori_loop(0, rpt // 16, zb, 0)

        pltpu.sync_copy(z2t_hbm.at[pl.ds(frow * NP, FPP * NP)], slab)
        plsc.subcore_barrier()

        row_base = (c * 2 + grp) * blocks_pt
        sh_base = grp * blocks_pt

        def outer(o, _):
            rb = row_base + o * 16
            pltpu.sync_copy(src2d.at[pl.ds(rb, 16)], sidx_v)
            pltpu.sync_copy(dst2d.at[pl.ds(rb, 16)], didx_v)

            def inner(jj, _):
                for u in range(2):
                    j = 2 * jj + u
                    sis = [sidx_v[j, pl.ds(g * 16, 16)] for g in range(8)]
                    dis = [didx_v[j, pl.ds(g * 16, 16)] for g in range(8)]
                    accs = [jnp.zeros((16,), jnp.float32) for _ in range(8)]
                    for p in range(FPP):
                        for g in range(8):
                            ws = plsc.load_gather(slab, [sis[g] + p * NP])
                            wd = plsc.load_gather(slab, [dis[g] + p * NP])
                            sb = plsc.bitcast(ws, jnp.bfloat16)
                            db = plsc.bitcast(wd, jnp.bfloat16)
                            pa, pb = plsc.unpack(
                                sb * db, format=plsc.PackFormat.INTERLEAVED,
                                preferred_element_type=jnp.float32)
                            accs[g] = accs[g] + pa + pb
                    for g in range(8):
                        scores_v[j, pl.ds(g * 16, 16)] = accs[g]
                return _

            lax.fori_loop(0, 8, inner, 0)
            rows = sh_base + o * 16 + lax.iota(jnp.int32, 16)
            pltpu.sync_copy(scores_v, score_sh.at[rows], add=True)
            return _

        lax.fori_loop(0, n_outer, outer, 0)
        plsc.subcore_barrier()

        pltpu.sync_copy(score_sh.at[pl.ds(s * rpt, rpt)],
                        out.at[pl.ds(c * rows_sh + s * rpt, rpt)])

    return k


def _mm1_body(x_ref, w1_ref, h_ref):
    h_ref[...] = jnp.dot(x_ref[...], w1_ref[...],
                         preferred_element_type=jnp.float32)


def _scale1_body(hist_ref, h_ref, g1_ref, dinv_ref):
    hp = hist_ref[0] + hist_ref[1]
    dinv = lax.rsqrt(hp + 1.0)
    g1_ref[...] = h_ref[...] * dinv[:N]
    dinv_ref[...] = dinv


def _enc2_body(agg_ref, g1_ref, dinv_ref, b1_ref, y1_ref):
    dinv = dinv_ref[pl.ds(0, N), :]
    agg = agg_ref[0, pl.ds(0, N), :] + agg_ref[1, pl.ds(0, N), :]
    z1 = (agg + g1_ref[...]) * dinv
    z1 = jnp.maximum(z1 + b1_ref[...], 0.0)
    y1_ref[...] = jnp.concatenate(
        [z1 * dinv, jnp.zeros((NP - N, HID), jnp.float32)], axis=0)


def _fin_body(agg_ref, y1_ref, dinv_ref, b2e_ref, b2o_ref, w2e_ref,
              w2o_ref, z2p_ref):
    t = (agg_ref[0] + agg_ref[1] + y1_ref[...]) * dinv_ref[...]
    ze = lax.dot_general(w2e_ref[...], t, (((0,), (1,)), ((), ())),
                         preferred_element_type=jnp.float32) + b2e_ref[...]
    zo = lax.dot_general(w2o_ref[...], t, (((0,), (1,)), ((), ())),
                         preferred_element_type=jnp.float32) + b2o_ref[...]
    ei = lax.bitcast_convert_type(ze.astype(jnp.bfloat16),
                                  jnp.uint16).astype(jnp.uint32)
    oi = lax.bitcast_convert_type(zo.astype(jnp.bfloat16),
                                  jnp.uint16).astype(jnp.uint32)
    z2p_ref[...] = lax.bitcast_convert_type(ei | (oi << 16), jnp.int32)


def _ceil_to(v, m):
    return ((v + m - 1) // m) * m


@jax.jit
def kernel(x, pos_edge_index, neg_edge_index, W1, b1, W2, b2):
    src = pos_edge_index[0].astype(jnp.int32)
    dst = pos_edge_index[1].astype(jnp.int32)

    H_OUTER, H_INNER = 10, 8
    eph = NW * H_OUTER * H_INNER * LANES
    pad_h = eph - E
    hpad = (jnp.arange(pad_h, dtype=jnp.int32) % 112) + N
    dst_h = jnp.concatenate([dst, hpad]).reshape(-1, LANES)
    histp = _hist_kernel(H_OUTER, H_INNER)(dst_h)

    h1 = pl.pallas_call(
        _mm1_body,
        out_shape=jax.ShapeDtypeStruct((N, HID), jnp.float32),
    )(x, W1)
    hist3 = histp.reshape(NC, NP, 1)
    g1, dinv = pl.pallas_call(
        _scale1_body,
        out_shape=(
            jax.ShapeDtypeStruct((N, HID), jnp.float32),
            jax.ShapeDtypeStruct((NP, 1), jnp.float32),
        ),
    )(hist3, h1)

    A_BLOCKS, A_NB = 10, 8
    epa = NW * A_BLOCKS * A_NB * LANES
    pad_a = epa - E
    pad_src = (jnp.arange(pad_a, dtype=jnp.int32) * 37) % N
    pad_dst = (jnp.arange(pad_a, dtype=jnp.int32) % 112) + N
    src_a = jnp.concatenate([src, pad_src]).reshape(-1, LANES)
    dst_a = jnp.concatenate([dst, pad_dst]).reshape(-1, LANES)

    agg1 = _agg_kernel(HID, A_BLOCKS, A_NB)(g1, src_a, dst_a)

    y1 = pl.pallas_call(
        _enc2_body,
        out_shape=jax.ShapeDtypeStruct((NP, HID), jnp.float32),
    )(agg1, g1, dinv, b1)

    agg2 = _agg_kernel(HID, A_BLOCKS, A_NB)(y1, src_a, dst_a)

    z2t = pl.pallas_call(
        _fin_body,
        out_shape=jax.ShapeDtypeStruct((OUT_C // 2, NP), jnp.int32),
    )(agg2, y1, dinv,
      b2[0::2].reshape(OUT_C // 2, 1), b2[1::2].reshape(OUT_C // 2, 1),
      W2[:, 0::2], W2[:, 1::2])

    D_OUTER = 80
    e2 = 2 * E
    epd = NC * 2 * D_OUTER * 16 * LANES
    pad_d = epd - e2
    pad_i = (jnp.arange(pad_d, dtype=jnp.int32) * 13) % N
    esrc = jnp.concatenate(
        [src, neg_edge_index[0].astype(jnp.int32), pad_i]).reshape(-1, LANES)
    edst = jnp.concatenate(
        [dst, neg_edge_index[1].astype(jnp.int32), pad_i]).reshape(-1, LANES)
    scores = _score_kernel(D_OUTER)(z2t.reshape(-1), esrc, edst).reshape(epd)

    return scores[:e2]


# speedup vs baseline: 1.0026x; 1.0026x over previous
import functools

import jax
import jax.numpy as jnp
from jax import lax
from jax.experimental import pallas as pl
from jax.experimental.pallas import tpu as pltpu
from jax.experimental.pallas import tpu_sc as plsc

N = 10000
NP = 10112
IN_C = 128
HID = 128
OUT_C = 64
E = 320000

NC = 2
NS = 16
NW = NC * NS
LANES = 128

_mesh = plsc.VectorSubcoreMesh(core_axis_name="c", subcore_axis_name="s")


def _hist_kernel(n_outer, n_inner):
    @functools.partial(
        pl.kernel,
        mesh=_mesh,
        out_type=jax.ShapeDtypeStruct((NC, NP), jnp.float32),
        scratch_types=[
            pltpu.VMEM((n_inner, LANES), jnp.int32),
            pltpu.VMEM((LANES,), jnp.float32),
            pltpu.VMEM_SHARED((NP,), jnp.float32),
            pltpu.VMEM((NP,), jnp.float32),
        ],
    )
    def k(dst2d, out, didx_v, ones_v, hist_sh, z_v):
        c = lax.axis_index("c")
        s = lax.axis_index("s")
        wid = c * NS + s
        for j in range(LANES // 16):
            ones_v[pl.ds(j * 16, 16)] = jnp.ones((16,), jnp.float32)

        @pl.when(s == 0)
        def _():
            def zb(i, _):
                z_v[pl.ds(i * 16, 16)] = jnp.zeros((16,), jnp.float32)
                return _
            lax.fori_loop(0, NP // 16, zb, 0)
            pltpu.sync_copy(z_v, hist_sh)

        plsc.subcore_barrier()

        row_base = wid * (n_outer * n_inner)

        def outer(o, _):
            pltpu.sync_copy(dst2d.at[pl.ds(row_base + o * n_inner, n_inner)],
                            didx_v)
            for j in range(n_inner):
                pltpu.sync_copy(ones_v, hist_sh.at[didx_v.at[j]], add=True)
            return _

        lax.fori_loop(0, n_outer, outer, 0)
        plsc.subcore_barrier()

        @pl.when(s == 0)
        def _():
            pltpu.sync_copy(hist_sh, out.at[c])

    return k


def _agg_kernel(C, n_blocks, nb):
    @functools.partial(
        pl.kernel,
        mesh=_mesh,
        out_type=jax.ShapeDtypeStruct((NC, NP, C), jnp.float32),
        scratch_types=[
            pltpu.VMEM((nb, LANES), jnp.int32),
            pltpu.VMEM((nb, LANES), jnp.int32),
            pltpu.VMEM((nb, LANES), jnp.int32),
            pltpu.VMEM((nb, LANES), jnp.int32),
            pltpu.VMEM((LANES, C), jnp.float32),
            pltpu.VMEM((LANES, C), jnp.float32),
            pltpu.VMEM_SHARED((NP, C), jnp.float32),
            pltpu.SemaphoreType.DMA,
            pltpu.SemaphoreType.DMA,
            pltpu.SemaphoreType.DMA,
            pltpu.SemaphoreType.DMA,
            pltpu.SemaphoreType.DMA,
            pltpu.SemaphoreType.DMA,
        ],
    )
    def k(g_hbm, src2d, dst2d, out,
          sidxA, didxA, sidxB, didxB, rows0, rows1, acc_sh,
          gs0, gs1, ss0, ss1, isA, isB):
        c = lax.axis_index("c")
        s = lax.axis_index("s")
        wid = c * NS + s
        rows_pt = NP // NS
        row0 = wid * n_blocks * nb

        def zrow(i, carry):
            r = i // (C // 16)
            col = (i % (C // 16)) * 16
            rows0[r, pl.ds(col, 16)] = jnp.zeros((16,), jnp.float32)
            return carry
        lax.fori_loop(0, LANES * (C // 16), zrow, 0)
        for kchunk in range(rows_pt // LANES):
            pltpu.sync_copy(
                rows0, acc_sh.at[pl.ds(s * rows_pt + kchunk * LANES, LANES)])
        rem = rows_pt % LANES
        if rem:
            pltpu.sync_copy(
                rows0.at[pl.ds(0, rem)],
                acc_sh.at[pl.ds(s * rows_pt + (rows_pt // LANES) * LANES,
                                rem)])

        def idx_load(bi, sidx, didx, isem):
            pltpu.async_copy(src2d.at[pl.ds(row0 + bi * nb, nb)], sidx, isem)
            pltpu.async_copy(dst2d.at[pl.ds(row0 + bi * nb, nb)], didx, isem)

        def idx_wait(sidx, didx, isem):
            pltpu.make_async_copy(src2d.at[pl.ds(0, nb)], sidx, isem).wait()
            pltpu.make_async_copy(dst2d.at[pl.ds(0, nb)], didx, isem).wait()

        idx_load(0, sidxA, didxA, isA)
        idx_load(1, sidxB, didxB, isB)
        plsc.subcore_barrier()

        def wait_g(rows, sem):
            pltpu.make_async_copy(g_hbm.at[sidxA.at[0]], rows, sem).wait()

        def wait_s(rows, sem):
            pltpu.make_async_copy(rows, acc_sh.at[didxA.at[0]], sem).wait()

        def run_block(sidx, didx):
            pltpu.async_copy(g_hbm.at[sidx.at[0]], rows0, gs0)

            def body(kk, inner):
                e0 = 2 * kk
                wait_g(rows0, gs0)

                @pl.when(kk >= 1)
                def _w1():
                    wait_s(rows1, ss1)
                pltpu.async_copy(g_hbm.at[sidx.at[e0 + 1]], rows1, gs1)
                pltpu.async_copy(rows0, acc_sh.at[didx.at[e0]], ss0,
                                 add=True)

                wait_g(rows1, gs1)
                wait_s(rows0, ss0)

                @pl.when(kk < nb // 2 - 1)
                def _g0():
                    pltpu.async_copy(g_hbm.at[sidx.at[e0 + 2]], rows0, gs0)
                pltpu.async_copy(rows1, acc_sh.at[didx.at[e0 + 1]], ss1,
                                 add=True)
                return inner

            lax.fori_loop(0, nb // 2, body, 0)
            wait_s(rows1, ss1)

        def blk2(m, carry):
            idx_wait(sidxA, didxA, isA)
            run_block(sidxA, didxA)

            @pl.when(m < n_blocks // 2 - 1)
            def _pa():
                idx_load(2 * m + 2, sidxA, didxA, isA)

            idx_wait(sidxB, didxB, isB)
            run_block(sidxB, didxB)

            @pl.when(m < n_blocks // 2 - 1)
            def _pb():
                idx_load(2 * m + 3, sidxB, didxB, isB)
            return carry

        lax.fori_loop(0, n_blocks // 2, blk2, 0)
        plsc.subcore_barrier()

        pltpu.sync_copy(acc_sh.at[pl.ds(s * rows_pt, rows_pt)],
                        out.at[c, pl.ds(s * rows_pt, rows_pt)])

    return k


def _score_kernel(n_outer):
    FPP = 4
    blocks_pt = n_outer * 16

    @functools.partial(
        pl.kernel,
        mesh=_mesh,
        out_type=jax.ShapeDtypeStruct((NC * 2 * blocks_pt, LANES),
                                      jnp.float32),
        compiler_params=pltpu.CompilerParams(needs_layout_passes=False),
        scratch_types=[
            pltpu.VMEM((FPP * NP,), jnp.int32),
            pltpu.VMEM((16, LANES), jnp.int32),
            pltpu.VMEM((16, LANES), jnp.int32),
            pltpu.VMEM((16, LANES), jnp.float32),
            pltpu.VMEM_SHARED((2 * blocks_pt, LANES), jnp.float32),
        ],
    )
    def k(z2t_hbm, src2d, dst2d, out,
          slab, sidx_v, didx_v, scores_v, score_sh):
        c = lax.axis_index("c")
        s = lax.axis_index("s")
        grp = s // 8
        frow = (s % 8) * FPP

        for j in range(16):
            for l in range(LANES // 16):
                scores_v[j, pl.ds(l * 16, 16)] = jnp.zeros((16,), jnp.float32)
        rows_sh = 2 * blocks_pt
        rpt = rows_sh // NS
        def zb(i, _):
            pltpu.sync_copy(scores_v, score_sh.at[pl.ds(s * rpt + i * 16, 16)])
            return _
        lax.fori_loop(0, rpt // 16, zb, 0)

        pltpu.sync_copy(z2t_hbm.at[pl.ds(frow * NP, FPP * NP)], slab)
        plsc.subcore_barrier()

        row_base = (c * 2 + grp) * blocks_pt
        sh_base = grp * blocks_pt

        def outer(o, _):
            rb = row_base + o * 16
            pltpu.sync_copy(src2d.at[pl.ds(rb, 16)], sidx_v)
            pltpu.sync_copy(dst2d.at[pl.ds(rb, 16)], didx_v)

            def inner(jj, _):
                for u in range(2):
                    j = 2 * jj + u
                    sis = [sidx_v[j, pl.ds(g * 16, 16)] for g in range(8)]
                    dis = [didx_v[j, pl.ds(g * 16, 16)] for g in range(8)]
                    accs = [jnp.zeros((16,), jnp.float32) for _ in range(8)]
                    for p in range(FPP):
                        for g in range(8):
                            ws = plsc.load_gather(slab, [sis[g] + p * NP])
                            wd = plsc.load_gather(slab, [dis[g] + p * NP])
                            sb = plsc.bitcast(ws, jnp.bfloat16)
                            db = plsc.bitcast(wd, jnp.bfloat16)
                            pa, pb = plsc.unpack(
                                sb * db, format=plsc.PackFormat.INTERLEAVED,
                                preferred_element_type=jnp.float32)
                            accs[g] = accs[g] + pa + pb
                    for g in range(8):
                        scores_v[j, pl.ds(g * 16, 16)] = accs[g]
                return _

            lax.fori_loop(0, 8, inner, 0)
            rows = sh_base + o * 16 + lax.iota(jnp.int32, 16)
            pltpu.sync_copy(scores_v, score_sh.at[rows], add=True)
            return _

        lax.fori_loop(0, n_outer, outer, 0)
        plsc.subcore_barrier()

        pltpu.sync_copy(score_sh.at[pl.ds(s * rpt, rpt)],
                        out.at[pl.ds(c * rows_sh + s * rpt, rpt)])

    return k


def _enc1_body(hist_ref, x_ref, w1_ref, g1_ref, dinv_ref):
    hp = hist_ref[0] + hist_ref[1]
    dinv = lax.rsqrt(hp + 1.0)
    h = jnp.dot(x_ref[...], w1_ref[...], preferred_element_type=jnp.float32)
    g1_ref[...] = h * dinv[:N]
    dinv_ref[...] = dinv


def _enc2_body(agg_ref, g1_ref, dinv_ref, b1_ref, y1_ref):
    dinv = dinv_ref[pl.ds(0, N), :]
    agg = agg_ref[0, pl.ds(0, N), :] + agg_ref[1, pl.ds(0, N), :]
    z1 = (agg + g1_ref[...]) * dinv
    z1 = jnp.maximum(z1 + b1_ref[...], 0.0)
    y1_ref[...] = jnp.concatenate(
        [z1 * dinv, jnp.zeros((NP - N, HID), jnp.float32)], axis=0)


def _fin_body(agg_ref, y1_ref, dinv_ref, b2e_ref, b2o_ref, w2e_ref,
              w2o_ref, z2p_ref):
    t = (agg_ref[0] + agg_ref[1] + y1_ref[...]) * dinv_ref[...]
    ze = lax.dot_general(w2e_ref[...], t, (((0,), (1,)), ((), ())),
                         preferred_element_type=jnp.float32) + b2e_ref[...]
    zo = lax.dot_general(w2o_ref[...], t, (((0,), (1,)), ((), ())),
                         preferred_element_type=jnp.float32) + b2o_ref[...]
    ei = lax.bitcast_convert_type(ze.astype(jnp.bfloat16),
                                  jnp.uint16).astype(jnp.uint32)
    oi = lax.bitcast_convert_type(zo.astype(jnp.bfloat16),
                                  jnp.uint16).astype(jnp.uint32)
    z2p_ref[...] = lax.bitcast_convert_type(ei | (oi << 16), jnp.int32)


def _ceil_to(v, m):
    return ((v + m - 1) // m) * m


@jax.jit
def kernel(x, pos_edge_index, neg_edge_index, W1, b1, W2, b2):
    src = pos_edge_index[0].astype(jnp.int32)
    dst = pos_edge_index[1].astype(jnp.int32)

    H_OUTER, H_INNER = 10, 8
    eph = NW * H_OUTER * H_INNER * LANES
    pad_h = eph - E
    hpad = (jnp.arange(pad_h, dtype=jnp.int32) % 112) + N
    dst_h = jnp.concatenate([dst, hpad]).reshape(-1, LANES)
    histp = _hist_kernel(H_OUTER, H_INNER)(dst_h)

    hist3 = histp.reshape(NC, NP, 1)
    g1, dinv = pl.pallas_call(
        _enc1_body,
        out_shape=(
            jax.ShapeDtypeStruct((N, HID), jnp.float32),
            jax.ShapeDtypeStruct((NP, 1), jnp.float32),
        ),
    )(hist3, x, W1)

    A_BLOCKS, A_NB = 10, 8
    epa = NW * A_BLOCKS * A_NB * LANES
    pad_a = epa - E
    pad_src = (jnp.arange(pad_a, dtype=jnp.int32) * 37) % N
    pad_dst = (jnp.arange(pad_a, dtype=jnp.int32) % 112) + N
    src_a = jnp.concatenate([src, pad_src]).reshape(-1, LANES)
    dst_a = jnp.concatenate([dst, pad_dst]).reshape(-1, LANES)

    agg1 = _agg_kernel(HID, A_BLOCKS, A_NB)(g1, src_a, dst_a)

    y1 = pl.pallas_call(
        _enc2_body,
        out_shape=jax.ShapeDtypeStruct((NP, HID), jnp.float32),
    )(agg1, g1, dinv, b1)

    agg2 = _agg_kernel(HID, A_BLOCKS, A_NB)(y1, src_a, dst_a)

    z2t = pl.pallas_call(
        _fin_body,
        out_shape=jax.ShapeDtypeStruct((OUT_C // 2, NP), jnp.int32),
    )(agg2, y1, dinv,
      b2[0::2].reshape(OUT_C // 2, 1), b2[1::2].reshape(OUT_C // 2, 1),
      W2[:, 0::2], W2[:, 1::2])

    D_OUTER = 80
    e2 = 2 * E
    epd = NC * 2 * D_OUTER * 16 * LANES
    pad_d = epd - e2
    pad_i = (jnp.arange(pad_d, dtype=jnp.int32) * 13) % N
    esrc = jnp.concatenate(
        [src, neg_edge_index[0].astype(jnp.int32), pad_i]).reshape(-1, LANES)
    edst = jnp.concatenate(
        [dst, neg_edge_index[1].astype(jnp.int32), pad_i]).reshape(-1, LANES)
    scores = _score_kernel(D_OUTER)(z2t.reshape(-1), esrc, edst).reshape(epd)

    return scores[:e2]


# speedup vs baseline: 1.3660x; 1.3624x over previous
import functools

import jax
import jax.numpy as jnp
from jax import lax
from jax.experimental import pallas as pl
from jax.experimental.pallas import tpu as pltpu
from jax.experimental.pallas import tpu_sc as plsc

N = 10000
NP = 10112
IN_C = 128
HID = 128
OUT_C = 64
E = 320000

NC = 2
NS = 16
NW = NC * NS
LANES = 128

_mesh = plsc.VectorSubcoreMesh(core_axis_name="c", subcore_axis_name="s")


def _hist_kernel(n_outer, n_inner):
    @functools.partial(
        pl.kernel,
        mesh=_mesh,
        out_type=jax.ShapeDtypeStruct((NC, NP), jnp.float32),
        scratch_types=[
            pltpu.VMEM((n_inner, LANES), jnp.int32),
            pltpu.VMEM((LANES,), jnp.float32),
            pltpu.VMEM_SHARED((NP,), jnp.float32),
            pltpu.VMEM((NP,), jnp.float32),
        ],
    )
    def k(dst2d, out, didx_v, ones_v, hist_sh, z_v):
        c = lax.axis_index("c")
        s = lax.axis_index("s")
        wid = c * NS + s
        for j in range(LANES // 16):
            ones_v[pl.ds(j * 16, 16)] = jnp.ones((16,), jnp.float32)

        @pl.when(s == 0)
        def _():
            def zb(i, _):
                z_v[pl.ds(i * 16, 16)] = jnp.zeros((16,), jnp.float32)
                return _
            lax.fori_loop(0, NP // 16, zb, 0)
            pltpu.sync_copy(z_v, hist_sh)

        plsc.subcore_barrier()

        row_base = wid * (n_outer * n_inner)

        def outer(o, _):
            pltpu.sync_copy(dst2d.at[pl.ds(row_base + o * n_inner, n_inner)],
                            didx_v)
            for j in range(n_inner):
                pltpu.sync_copy(ones_v, hist_sh.at[didx_v.at[j]], add=True)
            return _

        lax.fori_loop(0, n_outer, outer, 0)
        plsc.subcore_barrier()

        @pl.when(s == 0)
        def _():
            pltpu.sync_copy(hist_sh, out.at[c])

    return k


def _agg_kernel(C, n_blocks, nb):
    @functools.partial(
        pl.kernel,
        mesh=_mesh,
        out_type=jax.ShapeDtypeStruct((NC, NP, C), jnp.float32),
        scratch_types=[
            pltpu.VMEM((nb, LANES), jnp.int32),
            pltpu.VMEM((nb, LANES), jnp.int32),
            pltpu.VMEM((nb, LANES), jnp.int32),
            pltpu.VMEM((nb, LANES), jnp.int32),
            pltpu.VMEM((LANES, C), jnp.float32),
            pltpu.VMEM((LANES, C), jnp.float32),
            pltpu.VMEM_SHARED((NP, C), jnp.float32),
            pltpu.SemaphoreType.DMA,
            pltpu.SemaphoreType.DMA,
            pltpu.SemaphoreType.DMA,
            pltpu.SemaphoreType.DMA,
            pltpu.SemaphoreType.DMA,
            pltpu.SemaphoreType.DMA,
        ],
    )
    def k(g_hbm, src2d, dst2d, out,
          sidxA, didxA, sidxB, didxB, rows0, rows1, acc_sh,
          gs0, gs1, ss0, ss1, isA, isB):
        c = lax.axis_index("c")
        s = lax.axis_index("s")
        wid = c * NS + s
        rows_pt = NP // NS
        row0 = wid * n_blocks * nb

        def zrow(i, carry):
            r = i // (C // 16)
            col = (i % (C // 16)) * 16
            rows0[r, pl.ds(col, 16)] = jnp.zeros((16,), jnp.float32)
            return carry
        lax.fori_loop(0, LANES * (C // 16), zrow, 0)
        for kchunk in range(rows_pt // LANES):
            pltpu.sync_copy(
                rows0, acc_sh.at[pl.ds(s * rows_pt + kchunk * LANES, LANES)])
        rem = rows_pt % LANES
        if rem:
            pltpu.sync_copy(
                rows0.at[pl.ds(0, rem)],
                acc_sh.at[pl.ds(s * rows_pt + (rows_pt // LANES) * LANES,
                                rem)])

        def idx_load(bi, sidx, didx, isem):
            pltpu.async_copy(src2d.at[pl.ds(row0 + bi * nb, nb)], sidx, isem)
            pltpu.async_copy(dst2d.at[pl.ds(row0 + bi * nb, nb)], didx, isem)

        def idx_wait(sidx, didx, isem):
            pltpu.make_async_copy(src2d.at[pl.ds(0, nb)], sidx, isem).wait()
            pltpu.make_async_copy(dst2d.at[pl.ds(0, nb)], didx, isem).wait()

        idx_load(0, sidxA, didxA, isA)
        idx_load(1, sidxB, didxB, isB)
        plsc.subcore_barrier()

        def wait_g(rows, sem):
            pltpu.make_async_copy(g_hbm.at[sidxA.at[0]], rows, sem).wait()

        def wait_s(rows, sem):
            pltpu.make_async_copy(rows, acc_sh.at[didxA.at[0]], sem).wait()

        def run_block(sidx, didx):
            pltpu.async_copy(g_hbm.at[sidx.at[0]], rows0, gs0)

            def body(kk, inner):
                e0 = 2 * kk
                wait_g(rows0, gs0)

                @pl.when(kk >= 1)
                def _w1():
                    wait_s(rows1, ss1)
                pltpu.async_copy(g_hbm.at[sidx.at[e0 + 1]], rows1, gs1)
                pltpu.async_copy(rows0, acc_sh.at[didx.at[e0]], ss0,
                                 add=True)

                wait_g(rows1, gs1)
                wait_s(rows0, ss0)

                @pl.when(kk < nb // 2 - 1)
                def _g0():
                    pltpu.async_copy(g_hbm.at[sidx.at[e0 + 2]], rows0, gs0)
                pltpu.async_copy(rows1, acc_sh.at[didx.at[e0 + 1]], ss1,
                                 add=True)
                return inner

            lax.fori_loop(0, nb // 2, body, 0)
            wait_s(rows1, ss1)

        def blk2(m, carry):
            idx_wait(sidxA, didxA, isA)
            run_block(sidxA, didxA)

            @pl.when(m < n_blocks // 2 - 1)
            def _pa():
                idx_load(2 * m + 2, sidxA, didxA, isA)

            idx_wait(sidxB, didxB, isB)
            run_block(sidxB, didxB)

            @pl.when(m < n_blocks // 2 - 1)
            def _pb():
                idx_load(2 * m + 3, sidxB, didxB, isB)
            return carry

        lax.fori_loop(0, n_blocks // 2, blk2, 0)
        plsc.subcore_barrier()

        pltpu.sync_copy(acc_sh.at[pl.ds(s * rows_pt, rows_pt)],
                        out.at[c, pl.ds(s * rows_pt, rows_pt)])

    return k


def _score_kernel(n_outer):
    FPP = 4
    blocks_pt = n_outer * 16

    @functools.partial(
        pl.kernel,
        mesh=_mesh,
        out_type=jax.ShapeDtypeStruct((NC * 2 * blocks_pt, LANES),
                                      jnp.float32),
        compiler_params=pltpu.CompilerParams(needs_layout_passes=False),
        scratch_types=[
            pltpu.VMEM((FPP * NP,), jnp.int32),
            pltpu.VMEM((16, LANES), jnp.int32),
            pltpu.VMEM((16, LANES), jnp.int32),
            pltpu.VMEM((16, LANES), jnp.int32),
            pltpu.VMEM((16, LANES), jnp.int32),
            pltpu.VMEM((16, LANES), jnp.float32),
            pltpu.VMEM_SHARED((2 * blocks_pt, LANES), jnp.float32),
            pltpu.SemaphoreType.DMA,
            pltpu.SemaphoreType.DMA,
        ],
    )
    def k(z2t_hbm, src2d, dst2d, out,
          slab, sidxA, didxA, sidxB, didxB, scores_v, score_sh, iA, iB):
        c = lax.axis_index("c")
        s = lax.axis_index("s")
        grp = s // 8
        frow = (s % 8) * FPP

        for j in range(16):
            for l in range(LANES // 16):
                scores_v[j, pl.ds(l * 16, 16)] = jnp.zeros((16,), jnp.float32)
        rows_sh = 2 * blocks_pt
        rpt = rows_sh // NS
        def zb(i, _):
            pltpu.sync_copy(scores_v, score_sh.at[pl.ds(s * rpt + i * 16, 16)])
            return _
        lax.fori_loop(0, rpt // 16, zb, 0)

        pltpu.sync_copy(z2t_hbm.at[pl.ds(frow * NP, FPP * NP)], slab)
        plsc.subcore_barrier()

        row_base = (c * 2 + grp) * blocks_pt
        sh_base = grp * blocks_pt

        def idx_load(o, sidx, didx, isem):
            pltpu.async_copy(src2d.at[pl.ds(row_base + o * 16, 16)],
                             sidx, isem)
            pltpu.async_copy(dst2d.at[pl.ds(row_base + o * 16, 16)],
                             didx, isem)

        def idx_wait(sidx, didx, isem):
            pltpu.make_async_copy(src2d.at[pl.ds(0, 16)], sidx, isem).wait()
            pltpu.make_async_copy(dst2d.at[pl.ds(0, 16)], didx, isem).wait()

        idx_load(0, sidxA, didxA, iA)
        idx_load(1, sidxB, didxB, iB)

        def run_outer(o, sidx_v, didx_v):
            def inner(jj, _):
                for u in range(2):
                    j = 2 * jj + u
                    sis = [sidx_v[j, pl.ds(g * 16, 16)] for g in range(8)]
                    dis = [didx_v[j, pl.ds(g * 16, 16)] for g in range(8)]
                    accs = [jnp.zeros((16,), jnp.float32) for _ in range(8)]
                    for p in range(FPP):
                        for g in range(8):
                            ws = plsc.load_gather(slab, [sis[g] + p * NP])
                            wd = plsc.load_gather(slab, [dis[g] + p * NP])
                            sb = plsc.bitcast(ws, jnp.bfloat16)
                            db = plsc.bitcast(wd, jnp.bfloat16)
                            pa, pb = plsc.unpack(
                                sb * db, format=plsc.PackFormat.INTERLEAVED,
                                preferred_element_type=jnp.float32)
                            accs[g] = accs[g] + pa + pb
                    for g in range(8):
                        scores_v[j, pl.ds(g * 16, 16)] = accs[g]
                return _

            lax.fori_loop(0, 8, inner, 0)
            rows = sh_base + o * 16 + lax.iota(jnp.int32, 16)
            pltpu.sync_copy(scores_v, score_sh.at[rows], add=True)

        def outer2(m, carry):
            oA = 2 * m
            idx_wait(sidxA, didxA, iA)
            run_outer(oA, sidxA, didxA)

            @pl.when(oA + 2 < n_outer)
            def _la():
                idx_load(oA + 2, sidxA, didxA, iA)

            idx_wait(sidxB, didxB, iB)
            run_outer(oA + 1, sidxB, didxB)

            @pl.when(oA + 3 < n_outer)
            def _lb():
                idx_load(oA + 3, sidxB, didxB, iB)
            return carry

        lax.fori_loop(0, n_outer // 2, outer2, 0)
        plsc.subcore_barrier()

        pltpu.sync_copy(score_sh.at[pl.ds(s * rpt, rpt)],
                        out.at[pl.ds(c * rows_sh + s * rpt, rpt)])

    return k


def _enc1_body(hist_ref, x_ref, w1_ref, g1_ref, dinv_ref):
    hp = hist_ref[0] + hist_ref[1]
    dinv = lax.rsqrt(hp + 1.0)
    h = jnp.dot(x_ref[...], w1_ref[...], preferred_element_type=jnp.float32)
    g1_ref[...] = h * dinv[:N]
    dinv_ref[...] = dinv


def _enc2_body(agg_ref, g1_ref, dinv_ref, b1_ref, y1_ref):
    dinv = dinv_ref[pl.ds(0, N), :]
    agg = agg_ref[0, pl.ds(0, N), :] + agg_ref[1, pl.ds(0, N), :]
    z1 = (agg + g1_ref[...]) * dinv
    z1 = jnp.maximum(z1 + b1_ref[...], 0.0)
    y1_ref[...] = jnp.concatenate(
        [z1 * dinv, jnp.zeros((NP - N, HID), jnp.float32)], axis=0)


def _fin_body(agg_ref, y1_ref, dinv_ref, b2e_ref, b2o_ref, w2e_ref,
              w2o_ref, z2p_ref):
    t = (agg_ref[0] + agg_ref[1] + y1_ref[...]) * dinv_ref[...]
    ze = lax.dot_general(w2e_ref[...], t, (((0,), (1,)), ((), ())),
                         preferred_element_type=jnp.float32) + b2e_ref[...]
    zo = lax.dot_general(w2o_ref[...], t, (((0,), (1,)), ((), ())),
                         preferred_element_type=jnp.float32) + b2o_ref[...]
    ei = lax.bitcast_convert_type(ze.astype(jnp.bfloat16),
                                  jnp.uint16).astype(jnp.uint32)
    oi = lax.bitcast_convert_type(zo.astype(jnp.bfloat16),
                                  jnp.uint16).astype(jnp.uint32)
    z2p_ref[...] = lax.bitcast_convert_type(ei | (oi << 16), jnp.int32)


def _ceil_to(v, m):
    return ((v + m - 1) // m) * m


@jax.jit
def kernel(x, pos_edge_index, neg_edge_index, W1, b1, W2, b2):
    src = pos_edge_index[0].astype(jnp.int32)
    dst = pos_edge_index[1].astype(jnp.int32)

    H_OUTER, H_INNER = 10, 8
    eph = NW * H_OUTER * H_INNER * LANES
    pad_h = eph - E
    hpad = (jnp.arange(pad_h, dtype=jnp.int32) % 112) + N
    dst_h = jnp.concatenate([dst, hpad]).reshape(-1, LANES)
    histp = _hist_kernel(H_OUTER, H_INNER)(dst_h)

    hist3 = histp.reshape(NC, NP, 1)
    g1, dinv = pl.pallas_call(
        _enc1_body,
        out_shape=(
            jax.ShapeDtypeStruct((N, HID), jnp.float32),
            jax.ShapeDtypeStruct((NP, 1), jnp.float32),
        ),
    )(hist3, x, W1)

    A_BLOCKS, A_NB = 5, 16
    epa = NW * A_BLOCKS * A_NB * LANES
    pad_a = epa - E
    pad_src = (jnp.arange(pad_a, dtype=jnp.int32) * 37) % N
    pad_dst = (jnp.arange(pad_a, dtype=jnp.int32) % 112) + N
    src_a = jnp.concatenate([src, pad_src]).reshape(-1, LANES)
    dst_a = jnp.concatenate([dst, pad_dst]).reshape(-1, LANES)

    agg1 = _agg_kernel(HID, A_BLOCKS, A_NB)(g1, src_a, dst_a)

    y1 = pl.pallas_call(
        _enc2_body,
        out_shape=jax.ShapeDtypeStruct((NP, HID), jnp.float32),
    )(agg1, g1, dinv, b1)

    agg2 = _agg_kernel(HID, A_BLOCKS, A_NB)(y1, src_a, dst_a)

    z2t = pl.pallas_call(
        _fin_body,
        out_shape=jax.ShapeDtypeStruct((OUT_C // 2, NP), jnp.int32),
    )(agg2, y1, dinv,
      b2[0::2].reshape(OUT_C // 2, 1), b2[1::2].reshape(OUT_C // 2, 1),
      W2[:, 0::2], W2[:, 1::2])

    D_OUTER = 80
    e2 = 2 * E
    epd = NC * 2 * D_OUTER * 16 * LANES
    pad_d = epd - e2
    pad_i = (jnp.arange(pad_d, dtype=jnp.int32) * 13) % N
    esrc = jnp.concatenate(
        [src, neg_edge_index[0].astype(jnp.int32), pad_i]).reshape(-1, LANES)
    edst = jnp.concatenate(
        [dst, neg_edge_index[1].astype(jnp.int32), pad_i]).reshape(-1, LANES)
    scores = _score_kernel(D_OUTER)(z2t.reshape(-1), esrc, edst).reshape(epd)

    return scores[:e2]
